# Initial kernel scaffold; baseline (speedup 1.0000x reference)
import jax
import jax.numpy as jnp
from jax.experimental import pallas as pl


def kernel(x, edge_index, w_mul, lin1_W, lin1_b, mlp1_W1, mlp1_W2, mlp1_b2, lin2_W, lin2_b, mlp2_W1, mlp2_W2, mlp2_b2):
    raise NotImplementedError("write your pallas kernel here")


# speedup vs baseline: 8.0380x; 8.0380x over previous
import functools

import jax
import jax.numpy as jnp
from jax import lax
from jax.experimental import pallas as pl
from jax.experimental.pallas import tpu as pltpu
from jax.experimental.pallas import tpu_sc as plsc

N = 10000
E = 320000
F_IN = 128
H = 64
C = 7

NB = 10
BN = N // NB
NWORK = 32
EW = E // NWORK
B = 80
EPS = 1e-30


def _lrelu_pos(t):
    return jnp.where(t >= 0.0, t, 0.2 * t)


def _lrelu_neg(t):
    return jnp.where(t <= 0.0, t, 0.2 * t)


def _prep_body(c2d, w11, w12, w21d, w22d, par1, par2):
    cabs = jnp.max(jnp.abs(c2d[...]))
    up1 = jnp.dot(_lrelu_pos(w11[...]), w12[...], preferred_element_type=jnp.float32)
    un1 = jnp.dot(_lrelu_neg(w11[...]), w12[...], preferred_element_type=jnp.float32)
    m1 = cabs * jnp.maximum(jnp.abs(up1), jnp.abs(un1))
    par1[...] = jnp.concatenate([up1, un1, m1, jnp.zeros((5, H), jnp.float32)], axis=0)
    up2 = jnp.dot(_lrelu_pos(w21d[...]), w22d[...], preferred_element_type=jnp.float32)
    un2 = jnp.dot(_lrelu_neg(w21d[...]), w22d[...], preferred_element_type=jnp.float32)
    m2 = cabs * jnp.maximum(jnp.abs(up2), jnp.abs(un2))
    par2[...] = jnp.concatenate([up2, un2, m2, jnp.zeros((5, 16), jnp.float32)], axis=0)


def _prep(c, w11, w12, w21d, w22d):
    c2d = c.reshape(2500, 128)
    return pl.pallas_call(
        _prep_body,
        out_shape=(
            jax.ShapeDtypeStruct((8, H), jnp.float32),
            jax.ShapeDtypeStruct((8, 16), jnp.float32),
        ),
    )(c2d, w11, w12, w21d, w22d)


def _h1_body(x_ref, w_ref, b_ref, o_ref):
    o_ref[...] = (
        jnp.dot(x_ref[...], w_ref[...], preferred_element_type=jnp.float32)
        + b_ref[...]
    )


def _h1_matmul(x, w, b):
    return pl.pallas_call(
        _h1_body,
        grid=(NB,),
        in_specs=[
            pl.BlockSpec((BN, F_IN), lambda i: (i, 0)),
            pl.BlockSpec((F_IN, H), lambda i: (0, 0)),
            pl.BlockSpec((1, H), lambda i: (0, 0)),
        ],
        out_specs=pl.BlockSpec((BN, H), lambda i: (i, 0)),
        out_shape=jax.ShapeDtypeStruct((N, H), jnp.float32),
    )(x, w, b.reshape(1, H))


def _mid_body(ts_ref, w_ref, b_ref, o_ref):
    t = ts_ref[0] + ts_ref[1]
    num = t[:, :H]
    den = t[:, H:]
    o = num / (den + EPS)
    a = jnp.where(o > 0.0, o, jnp.exp(jnp.minimum(o, 0.0)) - 1.0)
    o_ref[...] = (
        jnp.dot(a, w_ref[...], preferred_element_type=jnp.float32) + b_ref[...]
    )


def _mid(ts1, w2d, b2d):
    return pl.pallas_call(
        _mid_body,
        grid=(NB,),
        in_specs=[
            pl.BlockSpec((2, BN, 2 * H), lambda i: (0, i, 0)),
            pl.BlockSpec((H, 16), lambda i: (0, 0)),
            pl.BlockSpec((1, 16), lambda i: (0, 0)),
        ],
        out_specs=pl.BlockSpec((BN, 16), lambda i: (i, 0)),
        out_shape=jax.ShapeDtypeStruct((N, 16), jnp.float32),
    )(ts1, w2d, b2d)


def _fin_body(ts_ref, o_ref):
    t = ts_ref[0] + ts_ref[1]
    num = t[:, 0:7]
    den = t[:, 8:15]
    o = num / (den + EPS)
    m = jnp.max(o, axis=1, keepdims=True)
    z = o - m
    o_ref[...] = z - jnp.log(jnp.sum(jnp.exp(z), axis=1, keepdims=True))


def _final(ts2):
    return pl.pallas_call(
        _fin_body,
        grid=(NB,),
        in_specs=[pl.BlockSpec((2, BN, 16), lambda i: (0, i, 0))],
        out_specs=pl.BlockSpec((BN, C), lambda i: (i, 0)),
        out_shape=jax.ShapeDtypeStruct((N, C), jnp.float32),
    )(ts2)


def _sc_edge_pass(h_tab, src, dst, c, par, dup):
    W = h_tab.shape[1]
    G = W // 16
    SW = 2 * W if dup else W
    ZR = 624
    TAIL = N - 16 * ZR
    zeros = jnp.zeros((ZR, SW), jnp.float32)
    mesh = plsc.VectorSubcoreMesh(core_axis_name="c", subcore_axis_name="s")

    @functools.partial(
        pl.kernel,
        out_type=jax.ShapeDtypeStruct((2, N, SW), jnp.float32),
        mesh=mesh,
        compiler_params=pltpu.CompilerParams(use_tc_tiling_on_sc=False),
        scratch_types=[
            pltpu.VMEM((B,), jnp.int32),
            pltpu.VMEM((B,), jnp.int32),
            pltpu.VMEM((B,), jnp.float32),
            pltpu.VMEM((B, W), jnp.float32),
            pltpu.VMEM((B, SW), jnp.float32),
            pltpu.VMEM((8, W), jnp.float32),
            pltpu.VMEM_SHARED((N, SW), jnp.float32),
            pltpu.SemaphoreType.DMA,
        ],
    )
    def kern(h_hbm, src_hbm, dst_hbm, c_hbm, par_hbm, zero_hbm, out_hbm,
             sidx, didx, cbuf, rows, stage, parv, ts_sh, sem):
        cid = lax.axis_index("c")
        sid = lax.axis_index("s")
        wid = sid * 2 + cid
        pltpu.sync_copy(zero_hbm, ts_sh.at[pl.ds(sid * ZR, ZR)])

        @pl.when(sid == 15)
        def _zero_tail():
            pltpu.sync_copy(zero_hbm.at[pl.ds(0, TAIL)],
                            ts_sh.at[pl.ds(16 * ZR, TAIL)])

        pltpu.sync_copy(par_hbm, parv)
        plsc.subcore_barrier()

        ups = [parv[0, pl.ds(16 * g, 16)] for g in range(G)]
        uns = [parv[1, pl.ds(16 * g, 16)] for g in range(G)]
        ms = [parv[2, pl.ds(16 * g, 16)] for g in range(G)]

        base = wid * EW

        def chunk_body(i, carry):
            off = base + i * B
            pltpu.sync_copy(src_hbm.at[pl.ds(off, B)], sidx)
            pltpu.sync_copy(dst_hbm.at[pl.ds(off, B)], didx)
            pltpu.sync_copy(c_hbm.at[pl.ds(off, B)], cbuf)
            pltpu.async_copy(h_hbm.at[sidx], rows, sem).wait()

            def edge_body(i2, carry2):
                cv = cbuf[pl.ds(16 * i2, 16)]
                cpv = jnp.maximum(cv, 0.0)
                cnv = jnp.minimum(cv, 0.0)
                for j in range(16):
                    e = 16 * i2 + j
                    ap = jnp.full((16,), cpv[j], jnp.float32)
                    an = jnp.full((16,), cnv[j], jnp.float32)
                    for g in range(G):
                        w = jnp.exp(ap * ups[g] + an * uns[g] - ms[g])
                        hrow = rows[e, pl.ds(16 * g, 16)]
                        stage[e, pl.ds(16 * g, 16)] = w * hrow
                        if dup:
                            stage[e, pl.ds(W + 16 * g, 16)] = w
                return carry2

            lax.fori_loop(0, B // 16, edge_body, 0)
            pltpu.sync_copy(stage, ts_sh.at[didx], add=True)
            return carry

        lax.fori_loop(0, EW // B, chunk_body, 0)
        plsc.subcore_barrier()
        pltpu.sync_copy(ts_sh.at[pl.ds(sid * ZR, ZR)],
                        out_hbm.at[cid, pl.ds(sid * ZR, ZR)])

        @pl.when(sid == 15)
        def _out_tail():
            pltpu.sync_copy(ts_sh.at[pl.ds(16 * ZR, TAIL)],
                            out_hbm.at[cid, pl.ds(16 * ZR, TAIL)])

    return kern(h_tab, src, dst, c, par, zeros)


def kernel(x, edge_index, w_mul, lin1_W, lin1_b, mlp1_W1, mlp1_W2, mlp1_b2,
           lin2_W, lin2_b, mlp2_W1, mlp2_W2, mlp2_b2):
    src = edge_index[0]
    dst = edge_index[1]
    c = w_mul[:, 0]

    w21d = jnp.zeros((1, 16), jnp.float32)
    w21d = w21d.at[0, 0:7].set(mlp2_W1[0]).at[0, 8:15].set(mlp2_W1[0])
    w22d = jnp.zeros((16, 16), jnp.float32)
    w22d = w22d.at[0:7, 0:7].set(mlp2_W2).at[8:15, 8:15].set(mlp2_W2)
    w2d = jnp.zeros((H, 16), jnp.float32).at[:, 0:7].set(lin2_W)
    b2d = jnp.zeros((1, 16), jnp.float32)
    b2d = b2d.at[0, 0:7].set(lin2_b).at[0, 8:15].set(1.0)

    par1, par2 = _prep(c, mlp1_W1, mlp1_W2, w21d, w22d)
    h1 = _h1_matmul(x, lin1_W, lin1_b)
    ts1 = _sc_edge_pass(h1, src, dst, c, par1, dup=True)
    h2 = _mid(ts1, w2d, b2d)
    ts2 = _sc_edge_pass(h2, src, dst, c, par2, dup=False)
    return _final(ts2)
